# baseline (speedup 1.0000x reference)
import functools
import jax
import jax.numpy as jnp
from jax import lax
from jax.experimental import pallas as pl
from jax.experimental.pallas import tpu as pltpu
from jax.experimental.pallas import tpu_sc as plsc

D_MODEL = 64
SCALE = 8.0
LANES = 16

NUM_CORES = 2
NUM_SUBCORES = 16
NW = NUM_CORES * NUM_SUBCORES

B_TOTAL = 4096 * 200
B_PER_W = B_TOTAL // NW
CHUNK = 1024
N_CHUNKS = B_PER_W // CHUNK

_MESH = plsc.VectorSubcoreMesh(core_axis_name="c", subcore_axis_name="s")


@functools.partial(
    pl.kernel,
    out_type=jax.ShapeDtypeStruct((B_TOTAL, D_MODEL), jnp.float32),
    mesh=_MESH,
    scratch_types=[
        pltpu.VMEM((CHUNK,), jnp.int32),
        pltpu.VMEM((CHUNK, D_MODEL), jnp.float32),
        pltpu.SemaphoreType.DMA,
    ],
    compiler_params=pltpu.CompilerParams(use_tc_tiling_on_sc=False),
)
def _emb_lookup(x_hbm, lut_hbm, out_hbm, idx_v, rows_v, sem):
    wid = lax.axis_index("s") * NUM_CORES + lax.axis_index("c")
    w_base = wid * B_PER_W

    def chunk_body(ci, carry):
        base = w_base + ci * CHUNK
        pltpu.sync_copy(x_hbm.at[pl.ds(base, CHUNK)], idx_v)
        pltpu.async_copy(lut_hbm.at[idx_v], rows_v, sem).wait()

        def row_body(r, c):
            for j in range(D_MODEL // LANES):
                s = pl.ds(j * LANES, LANES)
                rows_v[r, s] = rows_v[r, s] * SCALE
            return c

        lax.fori_loop(0, CHUNK, row_body, 0)
        pltpu.sync_copy(rows_v, out_hbm.at[pl.ds(base, CHUNK)])
        return carry

    lax.fori_loop(0, N_CHUNKS, chunk_body, 0)


def kernel(x, lut):
    out = _emb_lookup(x.reshape(-1), lut)
    return out.reshape(x.shape[0], x.shape[1], D_MODEL)


# speedup vs baseline: 1.1035x; 1.1035x over previous
import functools
import jax
import jax.numpy as jnp
from jax import lax
from jax.experimental import pallas as pl
from jax.experimental.pallas import tpu as pltpu
from jax.experimental.pallas import tpu_sc as plsc

D_MODEL = 64
SCALE = 8.0
LANES = 16

NUM_CORES = 2
NUM_SUBCORES = 16
NW = NUM_CORES * NUM_SUBCORES

B_TOTAL = 4096 * 200
B_PER_W = B_TOTAL // NW
CHUNK = 800
N_CHUNKS = B_PER_W // CHUNK

_MESH = plsc.VectorSubcoreMesh(core_axis_name="c", subcore_axis_name="s")


@functools.partial(
    pl.kernel,
    out_type=jax.ShapeDtypeStruct((B_TOTAL, D_MODEL), jnp.float32),
    mesh=_MESH,
    scratch_types=[
        pltpu.VMEM((N_CHUNKS, CHUNK), jnp.int32),
        pltpu.VMEM((CHUNK, D_MODEL), jnp.float32),
        pltpu.VMEM((CHUNK, D_MODEL), jnp.float32),
        pltpu.SemaphoreType.DMA,
        pltpu.SemaphoreType.DMA,
    ],
    compiler_params=pltpu.CompilerParams(use_tc_tiling_on_sc=False),
)
def _emb_lookup(x_hbm, lut_hbm, out_hbm, idx_all, rows0, rows1, sem0, sem1):
    wid = lax.axis_index("s") * NUM_CORES + lax.axis_index("c")
    w_base = wid * B_PER_W
    rows = (rows0, rows1)
    sems = (sem0, sem1)

    pltpu.sync_copy(x_hbm.at[wid], idx_all)
    pltpu.async_copy(lut_hbm.at[idx_all.at[0]], rows0, sem0)

    def scale_rows(buf):
        @plsc.parallel_loop(0, CHUNK, unroll=8)
        def _(r):
            for j in range(D_MODEL // LANES):
                s = pl.ds(j * LANES, LANES)
                buf[r, s] = buf[r, s] * SCALE

    @pl.loop(0, N_CHUNKS, step=2)
    def _(cc):
        for b in (0, 1):
            ci = cc + b

            @pl.when(ci + 1 < N_CHUNKS)
            def _():
                pltpu.async_copy(
                    lut_hbm.at[idx_all.at[ci + 1]], rows[1 - b], sems[1 - b]
                )

            pltpu.make_async_copy(
                lut_hbm.at[idx_all.at[ci]], rows[b], sems[b]
            ).wait()
            scale_rows(rows[b])
            pltpu.sync_copy(
                rows[b], out_hbm.at[pl.ds(w_base + ci * CHUNK, CHUNK)]
            )


def kernel(x, lut):
    out = _emb_lookup(x.reshape(NW, N_CHUNKS, CHUNK), lut)
    return out.reshape(x.shape[0], x.shape[1], D_MODEL)
